# baseline (speedup 1.0000x reference)
import functools

import jax
import jax.numpy as jnp
from jax import lax
from jax.experimental import pallas as pl
from jax.experimental.pallas import tpu as pltpu
from jax.experimental.pallas import tpu_sc as plsc

B, S, P, F = 4, 2048, 16, 128
D = P * F
ROWS = B * S
L = 16
NC, NS = 2, 16
NW = NC * NS
RPW = ROWS // NW
CH = 4
NCHUNK = RPW // CH
TABN = (P + 1) * D


def _compute_offsets(pad_v, offs_v):
    lanes = lax.iota(jnp.int32, L)

    @pl.loop(0, RPW // L)
    def _(g):
        row_idx = (g * L + lanes) * P
        acc = jnp.zeros((L,), jnp.int32)
        for k in range(P):
            col = plsc.load_gather(pad_v, [row_idx + k])
            acc = acc + jnp.where(col == 0, jnp.int32(1), jnp.int32(0))
        offs_v[pl.ds(g * L, L)] = acc * D


def _add_rows(cc, ibuf, obuf, offs_v, tab_v):
    lanes = lax.iota(jnp.int32, L)

    @pl.loop(0, CH)
    def _(r):
        rb = r * D
        row = cc * CH + r
        offv = plsc.load_gather(offs_v, [jnp.full((L,), row, jnp.int32)])
        base_idx = offv + lanes

        @plsc.parallel_loop(0, D // L, unroll=8)
        def _(j):
            trow = plsc.load_gather(tab_v, [base_idx + j * L])
            sl = pl.ds(rb + j * L, L)
            obuf[sl] = ibuf[sl] + trow


def _sc_body(in_hbm, pad_hbm, tab_hbm, out_hbm,
             tab_v, pad_v, offs_v, ib0, ib1, ob0, ob1,
             smi, smo,
             hi0, hi1, ti0, ti1, to0, to1, ho0, ho1):
    cid = lax.axis_index("c")
    sid = lax.axis_index("s")
    wid = sid * NC + cid
    base = wid * RPW

    pltpu.sync_copy(tab_hbm, tab_v)
    pltpu.sync_copy(pad_hbm.at[pl.ds(base * P, RPW * P)], pad_v)
    _compute_offsets(pad_v, offs_v)

    ibufs, obufs = (ib0, ib1), (ob0, ob1)
    smis = (smi.at[0], smi.at[1])
    smos = (smo.at[0], smo.at[1])
    hsi, tsi = (hi0, hi1), (ti0, ti1)
    tso, hso = (to0, to1), (ho0, ho1)

    def hbm_in(k):
        return in_hbm.at[pl.ds((base + k * CH) * D, CH * D)]

    def hbm_out(k):
        return out_hbm.at[pl.ds((base + k * CH) * D, CH * D)]

    pltpu.async_copy(hbm_in(0), smis[0].at[sid], hsi[0])
    pltpu.async_copy(hbm_in(1), smis[1].at[sid], hsi[1])

    @pl.loop(0, NCHUNK + 2, step=2)
    def _(c):
        for b in range(2):
            i = c + b
            bp = 1 - b

            @pl.when(i < NCHUNK)
            def _():
                pltpu.make_async_copy(hbm_in(i), smis[b].at[sid],
                                      hsi[b]).wait()
                pltpu.async_copy(smis[b].at[sid], ibufs[b], tsi[b])

            @pl.when(jnp.logical_and(i >= 1, i <= NCHUNK))
            def _():
                pltpu.make_async_copy(smis[bp].at[sid], ibufs[bp],
                                      tsi[bp]).wait()

                @pl.when(i + 1 < NCHUNK)
                def _():
                    pltpu.async_copy(hbm_in(i + 1), smis[bp].at[sid],
                                     hsi[bp])

                @pl.when(i >= 3)
                def _():
                    pltpu.make_async_copy(smos[bp].at[sid], hbm_out(i - 3),
                                          hso[bp]).wait()

                _add_rows(i - 1, ibufs[bp], obufs[bp], offs_v, tab_v)
                pltpu.async_copy(obufs[bp], smos[bp].at[sid], tso[bp])

            @pl.when(jnp.logical_and(i >= 2, i <= NCHUNK + 1))
            def _():
                pltpu.make_async_copy(obufs[b], smos[b].at[sid],
                                      tso[b]).wait()
                pltpu.async_copy(smos[b].at[sid], hbm_out(i - 2), hso[b])

    pltpu.make_async_copy(smos[0].at[sid], hbm_out(NCHUNK - 2),
                          hso[0]).wait()
    pltpu.make_async_copy(smos[1].at[sid], hbm_out(NCHUNK - 1),
                          hso[1]).wait()


@functools.partial(jax.jit, static_argnums=())
def _run(flat_in, flat_pad, flat_tab):
    mesh = plsc.VectorSubcoreMesh(core_axis_name="c", subcore_axis_name="s",
                                  num_cores=NC, num_subcores=NS)
    f = pl.kernel(
        _sc_body,
        out_type=jax.ShapeDtypeStruct((ROWS * D,), jnp.float32),
        mesh=mesh,
        compiler_params=pltpu.CompilerParams(needs_layout_passes=False),
        scratch_types=[
            pltpu.VMEM((TABN,), jnp.float32),
            pltpu.VMEM((RPW * P,), jnp.int32),
            pltpu.VMEM((RPW,), jnp.int32),
            pltpu.VMEM((CH * D,), jnp.float32),
            pltpu.VMEM((CH * D,), jnp.float32),
            pltpu.VMEM((CH * D,), jnp.float32),
            pltpu.VMEM((CH * D,), jnp.float32),
            pltpu.VMEM_SHARED((2, NS, CH * D), jnp.float32),
            pltpu.VMEM_SHARED((2, NS, CH * D), jnp.float32),
            pltpu.SemaphoreType.DMA,
            pltpu.SemaphoreType.DMA,
            pltpu.SemaphoreType.DMA,
            pltpu.SemaphoreType.DMA,
            pltpu.SemaphoreType.DMA,
            pltpu.SemaphoreType.DMA,
            pltpu.SemaphoreType.DMA,
            pltpu.SemaphoreType.DMA,
        ],
    )
    return f(flat_in, flat_pad, flat_tab)


def kernel(inputs, paddings, table):
    flat_in = inputs.reshape(ROWS * D)
    flat_pad = paddings.reshape(ROWS * P)
    flat_tab = table.reshape(TABN)
    out = _run(flat_in, flat_pad, flat_tab)
    return out.reshape(B, S, D)
